# baseline (speedup 1.0000x reference)
import functools
import numpy as np
import jax
import jax.numpy as jnp
from jax.experimental import pallas as pl
from jax.experimental.pallas import tpu as pltpu

_QUANTILES = np.float32([0.1, 0.25, 0.5, 0.75, 0.9])
_NQ = 5
_INT_MIN = np.int32(-(2**31))
_INT_MAX = np.int32(2**31 - 1)


def _quantile_body(ranks, w_lo, w_hi, x_ref, o_ref):
    x = x_ref[...]
    i = jax.lax.bitcast_convert_type(x, jnp.int32)
    key = i ^ (jax.lax.shift_right_arithmetic(i, 31) & jnp.int32(0x7FFFFFFF))

    br = x.shape[0]
    n = x.shape[1]
    ones = jnp.ones((n, 1), jnp.float32)

    def step(it, r):
        bit = jax.lax.shift_left(jnp.int32(1), jnp.int32(31) - it)
        t = r + bit
        p = _INT_MIN + t
        cols = []
        for q in range(_NQ):
            m = jnp.where(key < p[:, q:q + 1], 1.0, 0.0)
            c = jax.lax.dot_general(m, ones, (((1,), (0,)), ((), ())),
                                    preferred_element_type=jnp.float32)
            cols.append(jnp.where(c <= float(ranks[q]), t[:, q:q + 1],
                                  r[:, q:q + 1]))
        return jnp.concatenate(cols, axis=1)

    r = jax.lax.fori_loop(0, 32, step, jnp.zeros((br, _NQ), jnp.int32))
    rkey = _INT_MIN + r

    outs = []
    for q in range(_NQ):
        lo = rkey[:, q:q + 1]
        le = jnp.sum((key <= lo).astype(jnp.int32), axis=1, keepdims=True)
        gt_min = jnp.min(jnp.where(key > lo, key, _INT_MAX), axis=1,
                         keepdims=True)
        hi = jnp.where(le >= ranks[q] + 2, lo, gt_min)
        f_lo = _key_to_f32(lo)
        f_hi = _key_to_f32(hi)
        outs.append(f_lo * w_lo[q] + f_hi * w_hi[q])
    o_ref[...] = jnp.concatenate(outs, axis=1)


def _key_to_f32(key):
    i = jnp.where(key >= 0, key, key ^ jnp.int32(0x7FFFFFFF))
    return jax.lax.bitcast_convert_type(i, jnp.float32)


@jax.jit
def kernel(input):
    b, t, n = input.shape
    rows = b * t
    x = input.reshape(rows, n)

    idxf = _QUANTILES * np.float32(n - 1)
    ranks = np.floor(idxf).astype(np.int32)
    w_hi = (idxf - ranks).astype(np.float32)
    w_lo = (np.float32(1.0) - w_hi).astype(np.float32)

    br = 128 if rows % 128 == 0 else rows
    grid = rows // br
    out = pl.pallas_call(
        functools.partial(_quantile_body, ranks, w_lo, w_hi),
        grid=(grid,),
        in_specs=[pl.BlockSpec((br, n), lambda g: (g, 0))],
        out_specs=pl.BlockSpec((br, _NQ), lambda g: (g, 0)),
        out_shape=jax.ShapeDtypeStruct((rows, _NQ), jnp.float32),
        compiler_params=pltpu.CompilerParams(
            dimension_semantics=("arbitrary",),
        ),
    )(x)
    return out.reshape(b, t, _NQ)


# speedup vs baseline: 1.2708x; 1.2708x over previous
import functools
import numpy as np
import jax
import jax.numpy as jnp
from jax.experimental import pallas as pl
from jax.experimental.pallas import tpu as pltpu

_QUANTILES = np.float32([0.1, 0.25, 0.5, 0.75, 0.9])
_NQ = 5
_INT_MIN = np.int32(-(2**31))
_INT_MAX = np.int32(2**31 - 1)


def _quantile_body(ranks, w_lo, w_hi, x_ref, o_ref):
    x = x_ref[...]
    i = jax.lax.bitcast_convert_type(x, jnp.int32)
    key = i ^ (jax.lax.shift_right_arithmetic(i, 31) & jnp.int32(0x7FFFFFFF))

    br = x.shape[0]

    def step(it, r):
        bit = jax.lax.shift_left(jnp.int32(1), jnp.int32(31) - it)
        t = r + bit
        p = _INT_MIN + t
        m01 = (jnp.where(key < p[:, 0:1], 1, 0)
               + jnp.where(key < p[:, 1:2], 1 << 16, 0))
        m23 = (jnp.where(key < p[:, 2:3], 1, 0)
               + jnp.where(key < p[:, 3:4], 1 << 16, 0))
        m4 = jnp.where(key < p[:, 4:5], 1, 0)
        s01 = jnp.sum(m01, axis=1, keepdims=True)
        s23 = jnp.sum(m23, axis=1, keepdims=True)
        s4 = jnp.sum(m4, axis=1, keepdims=True)
        counts = [s01 & 0xFFFF, jax.lax.shift_right_logical(s01, 16),
                  s23 & 0xFFFF, jax.lax.shift_right_logical(s23, 16), s4]
        cols = [jnp.where(counts[q] <= int(ranks[q]), t[:, q:q + 1],
                          r[:, q:q + 1]) for q in range(_NQ)]
        return jnp.concatenate(cols, axis=1)

    r = jax.lax.fori_loop(0, 32, step, jnp.zeros((br, _NQ), jnp.int32))
    rkey = _INT_MIN + r

    outs = []
    for q in range(_NQ):
        lo = rkey[:, q:q + 1]
        le = jnp.sum((key <= lo).astype(jnp.int32), axis=1, keepdims=True)
        gt_min = jnp.min(jnp.where(key > lo, key, _INT_MAX), axis=1,
                         keepdims=True)
        hi = jnp.where(le >= ranks[q] + 2, lo, gt_min)
        f_lo = _key_to_f32(lo)
        f_hi = _key_to_f32(hi)
        outs.append(f_lo * w_lo[q] + f_hi * w_hi[q])
    o_ref[...] = jnp.concatenate(outs, axis=1)


def _key_to_f32(key):
    i = jnp.where(key >= 0, key, key ^ jnp.int32(0x7FFFFFFF))
    return jax.lax.bitcast_convert_type(i, jnp.float32)


@jax.jit
def kernel(input):
    b, t, n = input.shape
    rows = b * t
    x = input.reshape(rows, n)

    idxf = _QUANTILES * np.float32(n - 1)
    ranks = np.floor(idxf).astype(np.int32)
    w_hi = (idxf - ranks).astype(np.float32)
    w_lo = (np.float32(1.0) - w_hi).astype(np.float32)

    br = 128 if rows % 128 == 0 else rows
    grid = rows // br
    out = pl.pallas_call(
        functools.partial(_quantile_body, ranks, w_lo, w_hi),
        grid=(grid,),
        in_specs=[pl.BlockSpec((br, n), lambda g: (g, 0))],
        out_specs=pl.BlockSpec((br, _NQ), lambda g: (g, 0)),
        out_shape=jax.ShapeDtypeStruct((rows, _NQ), jnp.float32),
        compiler_params=pltpu.CompilerParams(
            dimension_semantics=("arbitrary",),
        ),
    )(x)
    return out.reshape(b, t, _NQ)
